# Initial kernel scaffold; baseline (speedup 1.0000x reference)
import jax
import jax.numpy as jnp
from jax.experimental import pallas as pl


def kernel(x, k_weights, k_indices, feature_density, activated_in, feature_means, feature_square_means, avg_norm, n_steps):
    raise NotImplementedError("write your pallas kernel here")


# speedup vs baseline: 41.7481x; 41.7481x over previous
import functools

import jax
import jax.numpy as jnp
from jax import lax
from jax.experimental import pallas as pl
from jax.experimental.pallas import tpu as pltpu
from jax.experimental.pallas import tpu_sc as plsc

D_MODEL = 1024
N_FEATURES = 131072
BATCH = 4096
K = 64
TOTAL_IDX = BATCH * K
DEATH_THRESHOLD = 0.01

NC = 2
NS = 16
LANES = 16
PER_TILE_IDX = TOTAL_IDX // NS
CHUNK = 128
N_CHUNKS = PER_TILE_IDX // CHUNK
BINS_PER_TILE = N_FEATURES // (NC * NS)
HALF_BINS = N_FEATURES // NC
ZEROS_N = N_FEATURES // NS

_NBLK = 8
_BLK = BATCH // _NBLK


def _tc_body(params, x_ref, fm_ref, fsm_ref, means_out, sq_out, norm_out):
    i = pl.program_id(0)
    blk = x_ref[...]
    sq = blk * blk
    csum = jnp.sum(blk, axis=0, keepdims=True)
    cssum = jnp.sum(sq, axis=0, keepdims=True)
    nsum = jnp.sum(jnp.sqrt(jnp.sum(sq, axis=1, keepdims=True)))

    @pl.when(i == 0)
    def _():
        means_out[...] = csum
        sq_out[...] = cssum
        norm_out[0, 0] = nsum

    @pl.when(i > 0)
    def _():
        means_out[...] += csum
        sq_out[...] += cssum
        norm_out[0, 0] += nsum

    @pl.when(i == _NBLK - 1)
    def _():
        wf = params[0]
        scale = params[1]
        means_out[...] = fm_ref[...] * wf + means_out[...] * scale
        sq_out[...] = fsm_ref[...] * wf + sq_out[...] * scale
        norm_out[0, 0] = params[2] * wf + norm_out[0, 0] * scale


def _tc_stats(params_tc, x, feature_means, feature_square_means):
    return pl.pallas_call(
        _tc_body,
        grid=(_NBLK,),
        in_specs=[
            pl.BlockSpec(memory_space=pltpu.SMEM),
            pl.BlockSpec((_BLK, D_MODEL), lambda i: (i, 0)),
            pl.BlockSpec((1, D_MODEL), lambda i: (0, 0)),
            pl.BlockSpec((1, D_MODEL), lambda i: (0, 0)),
        ],
        out_specs=[
            pl.BlockSpec((1, D_MODEL), lambda i: (0, 0)),
            pl.BlockSpec((1, D_MODEL), lambda i: (0, 0)),
            pl.BlockSpec(memory_space=pltpu.SMEM),
        ],
        out_shape=[
            jax.ShapeDtypeStruct((1, D_MODEL), jnp.float32),
            jax.ShapeDtypeStruct((1, D_MODEL), jnp.float32),
            jax.ShapeDtypeStruct((1, 1), jnp.float32),
        ],
        compiler_params=pltpu.CompilerParams(
            dimension_semantics=("arbitrary",)),
    )(params_tc, x, feature_means.reshape(1, D_MODEL),
      feature_square_means.reshape(1, D_MODEL))


def _sc_body(prm_hbm, idx_hbm, w_hbm, fd_hbm, ai_hbm,
             fd_out, ai_out,
             hist_cnt, hist_nd,
             idx_v, w_v, nd_v, ones_v, zero_v,
             fd_in_v, ai_in_v, cnt_rd, nd_rd, fdo_v, aio_v, prm_v):
    c = lax.axis_index("c")
    s = lax.axis_index("s")

    pltpu.sync_copy(idx_hbm.at[s], idx_v)
    pltpu.sync_copy(w_hbm.at[s], w_v)
    pltpu.sync_copy(prm_hbm, prm_v)

    one16 = jnp.full((LANES,), 1, dtype=jnp.int32)
    for l in range(CHUNK // LANES):
        ones_v[pl.ds(l * LANES, LANES)] = one16

    z16 = jnp.zeros((LANES,), dtype=jnp.int32)

    @pl.loop(0, ZEROS_N // LANES)
    def _(i):
        zero_v[pl.ds(i * LANES, LANES)] = z16

    thr16 = jnp.full((LANES,), DEATH_THRESHOLD, dtype=jnp.float32)

    @pl.loop(0, N_CHUNKS)
    def _(j):
        for l in range(CHUNK // LANES):
            w16 = w_v[j, pl.ds(l * LANES, LANES)]
            nd_v[j, pl.ds(l * LANES, LANES)] = jnp.where(w16 > thr16, one16, z16)

    zbase = s * ZEROS_N
    pltpu.sync_copy(zero_v, hist_cnt.at[pl.ds(zbase, ZEROS_N)])
    pltpu.sync_copy(zero_v, hist_nd.at[pl.ds(zbase, ZEROS_N)])
    plsc.subcore_barrier()

    @pl.loop(0, N_CHUNKS)
    def _(j):
        pltpu.sync_copy(ones_v, hist_cnt.at[idx_v.at[j]], add=True)
        pltpu.sync_copy(nd_v.at[j], hist_nd.at[idx_v.at[j]], add=True)

    plsc.subcore_barrier()

    gbase = c * HALF_BINS + s * BINS_PER_TILE
    pltpu.sync_copy(hist_cnt.at[pl.ds(gbase, BINS_PER_TILE)], cnt_rd)
    pltpu.sync_copy(hist_nd.at[pl.ds(gbase, BINS_PER_TILE)], nd_rd)
    pltpu.sync_copy(fd_hbm.at[pl.ds(gbase, BINS_PER_TILE)], fd_in_v)
    pltpu.sync_copy(ai_hbm.at[pl.ds(gbase, BINS_PER_TILE)], ai_in_v)

    wf16 = prm_v[0, :]
    scale16 = prm_v[1, :]

    zero16 = jnp.zeros((LANES,), dtype=jnp.int32)

    @pl.loop(0, BINS_PER_TILE // LANES)
    def _(i):
        sl = pl.ds(i * LANES, LANES)
        fd16 = fd_in_v[sl]
        cnt16 = cnt_rd[sl].astype(jnp.float32)
        fdo_v[sl] = fd16 * wf16 + cnt16 * scale16
        nd16 = nd_rd[sl]
        ai16 = ai_in_v[sl]
        aio_v[sl] = jnp.where(nd16 > zero16, zero16, ai16 + one16)

    pltpu.sync_copy(fdo_v, fd_out.at[pl.ds(gbase, BINS_PER_TILE)])
    pltpu.sync_copy(aio_v, ai_out.at[pl.ds(gbase, BINS_PER_TILE)])


_sc_hist = pl.kernel(
    _sc_body,
    out_type=[
        jax.ShapeDtypeStruct((N_FEATURES,), jnp.float32),
        jax.ShapeDtypeStruct((N_FEATURES,), jnp.int32),
    ],
    mesh=plsc.VectorSubcoreMesh(core_axis_name="c", subcore_axis_name="s"),
    scratch_types=[
        pltpu.VMEM_SHARED((N_FEATURES,), jnp.int32),
        pltpu.VMEM_SHARED((N_FEATURES,), jnp.int32),
        pltpu.VMEM((N_CHUNKS, CHUNK), jnp.int32),
        pltpu.VMEM((N_CHUNKS, CHUNK), jnp.float32),
        pltpu.VMEM((N_CHUNKS, CHUNK), jnp.int32),
        pltpu.VMEM((CHUNK,), jnp.int32),
        pltpu.VMEM((ZEROS_N,), jnp.int32),
        pltpu.VMEM((BINS_PER_TILE,), jnp.float32),
        pltpu.VMEM((BINS_PER_TILE,), jnp.int32),
        pltpu.VMEM((BINS_PER_TILE,), jnp.int32),
        pltpu.VMEM((BINS_PER_TILE,), jnp.int32),
        pltpu.VMEM((BINS_PER_TILE,), jnp.float32),
        pltpu.VMEM((BINS_PER_TILE,), jnp.int32),
        pltpu.VMEM((2, LANES), jnp.float32),
    ],
)


def kernel(x, k_weights, k_indices, feature_density, activated_in,
           feature_means, feature_square_means, avg_norm, n_steps):
    wf = (n_steps / (n_steps + 1)).astype(jnp.float32)
    nwf = (1 / (n_steps + 1)).astype(jnp.float32)

    params_tc = jnp.stack([wf, nwf / BATCH, avg_norm])
    means2, sq2, norm11 = _tc_stats(params_tc, x, feature_means,
                                    feature_square_means)

    idx3 = k_indices.reshape(NS, N_CHUNKS, CHUNK)
    w3 = k_weights.reshape(NS, N_CHUNKS, CHUNK)
    ai_i = lax.bitcast_convert_type(activated_in, jnp.int32)
    prm2 = jnp.stack([jnp.full((LANES,), wf, dtype=jnp.float32),
                      jnp.full((LANES,), nwf / BATCH, dtype=jnp.float32)])
    fd_new, ai_new_i = _sc_hist(prm2, idx3, w3, feature_density, ai_i)
    ai_new = lax.bitcast_convert_type(ai_new_i, jnp.uint32)

    return (norm11[0, 0], means2.reshape(D_MODEL), sq2.reshape(D_MODEL),
            fd_new, ai_new)


# speedup vs baseline: 62.6239x; 1.5000x over previous
import functools

import jax
import jax.numpy as jnp
from jax import lax
from jax.experimental import pallas as pl
from jax.experimental.pallas import tpu as pltpu
from jax.experimental.pallas import tpu_sc as plsc

D_MODEL = 1024
N_FEATURES = 131072
BATCH = 4096
K = 64
TOTAL_IDX = BATCH * K
DEATH_THRESHOLD = 0.01

NC = 2
NS = 16
LANES = 16
PER_TILE_IDX = TOTAL_IDX // NS
BINS2 = N_FEATURES // NS
ZEROS_N = N_FEATURES // NS

_NBLK = 8
_BLK = BATCH // _NBLK


def _tc_body(params, x_ref, fm_ref, fsm_ref, means_out, sq_out, norm_out):
    i = pl.program_id(0)
    blk = x_ref[...]
    sq = blk * blk
    csum = jnp.sum(blk, axis=0, keepdims=True)
    cssum = jnp.sum(sq, axis=0, keepdims=True)
    nsum = jnp.sum(jnp.sqrt(jnp.sum(sq, axis=1, keepdims=True)))

    @pl.when(i == 0)
    def _():
        means_out[...] = csum
        sq_out[...] = cssum
        norm_out[0, 0] = nsum

    @pl.when(i > 0)
    def _():
        means_out[...] += csum
        sq_out[...] += cssum
        norm_out[0, 0] += nsum

    @pl.when(i == _NBLK - 1)
    def _():
        wf = params[0]
        scale = params[1]
        means_out[...] = fm_ref[...] * wf + means_out[...] * scale
        sq_out[...] = fsm_ref[...] * wf + sq_out[...] * scale
        norm_out[0, 0] = params[2] * wf + norm_out[0, 0] * scale


def _tc_stats(params_tc, x, feature_means, feature_square_means):
    return pl.pallas_call(
        _tc_body,
        grid=(_NBLK,),
        in_specs=[
            pl.BlockSpec(memory_space=pltpu.SMEM),
            pl.BlockSpec((_BLK, D_MODEL), lambda i: (i, 0)),
            pl.BlockSpec((1, D_MODEL), lambda i: (0, 0)),
            pl.BlockSpec((1, D_MODEL), lambda i: (0, 0)),
        ],
        out_specs=[
            pl.BlockSpec((1, D_MODEL), lambda i: (0, 0)),
            pl.BlockSpec((1, D_MODEL), lambda i: (0, 0)),
            pl.BlockSpec(memory_space=pltpu.SMEM),
        ],
        out_shape=[
            jax.ShapeDtypeStruct((1, D_MODEL), jnp.float32),
            jax.ShapeDtypeStruct((1, D_MODEL), jnp.float32),
            jax.ShapeDtypeStruct((1, 1), jnp.float32),
        ],
        compiler_params=pltpu.CompilerParams(
            dimension_semantics=("arbitrary",)),
    )(params_tc, x, feature_means.reshape(1, D_MODEL),
      feature_square_means.reshape(1, D_MODEL))


def _sc_body(prm_hbm, idx_hbm, w_hbm, fd_hbm, ai_hbm, ones_hbm, zeros_hbm,
             fd_out, ai_out,
             hist,
             idx_v, w_v, nd_v, ones_v,
             fd_in_v, ai_in_v, h_rd, fdo_v, aio_v, prm_v,
             semi, semw, semo, semz, semf, sem0):
    c = lax.axis_index("c")
    s = lax.axis_index("s")

    ibase = s * PER_TILE_IDX
    d_idx = pltpu.async_copy(idx_hbm.at[pl.ds(ibase, PER_TILE_IDX)], idx_v,
                             semi)
    zbase = s * ZEROS_N
    d_z = pltpu.async_copy(zeros_hbm, hist.at[pl.ds(zbase, ZEROS_N)], semz)
    gbase = s * BINS2

    one16 = jnp.full((LANES,), 1, dtype=jnp.int32)
    z16 = jnp.zeros((LANES,), dtype=jnp.int32)
    thr16 = jnp.full((LANES,), DEATH_THRESHOLD, dtype=jnp.float32)
    zu16 = jnp.zeros((LANES,), dtype=jnp.uint32)
    ou16 = jnp.full((LANES,), 1, dtype=jnp.uint32)

    @pl.when(c == 0)
    def _():
        d_ones = pltpu.async_copy(ones_hbm, ones_v, semo)
        d_fd = pltpu.async_copy(fd_hbm.at[pl.ds(gbase, BINS2)], fd_in_v,
                                semf)
        pltpu.sync_copy(prm_hbm, prm_v)
        d_z.wait()
        plsc.subcore_barrier()
        d_idx.wait()
        d_ones.wait()
        pltpu.async_copy(ones_v, hist.at[idx_v], sem0, add=True).wait()
        plsc.subcore_barrier()
        pltpu.sync_copy(hist.at[pl.ds(gbase, BINS2)], h_rd)
        d_fd.wait()
        wf16 = prm_v[0, :]
        scale16 = prm_v[1, :]

        @pl.loop(0, BINS2 // LANES, unroll=8)
        def _(i):
            sl = pl.ds(i * LANES, LANES)
            cnt16 = h_rd[sl].astype(jnp.float32)
            fdo_v[sl] = fd_in_v[sl] * wf16 + cnt16 * scale16

        pltpu.sync_copy(fdo_v, fd_out.at[pl.ds(gbase, BINS2)])

    @pl.when(c == 1)
    def _():
        d_w = pltpu.async_copy(w_hbm.at[pl.ds(ibase, PER_TILE_IDX)], w_v,
                               semw)
        d_ai = pltpu.async_copy(ai_hbm.at[pl.ds(gbase, BINS2)], ai_in_v,
                                semf)
        d_w.wait()

        @pl.loop(0, PER_TILE_IDX // LANES, unroll=8)
        def _(j):
            sl = pl.ds(j * LANES, LANES)
            w16 = w_v[sl]
            nd_v[sl] = jnp.where(w16 > thr16, one16, z16)

        d_z.wait()
        plsc.subcore_barrier()
        d_idx.wait()
        pltpu.async_copy(nd_v, hist.at[idx_v], sem0, add=True).wait()
        plsc.subcore_barrier()
        pltpu.sync_copy(hist.at[pl.ds(gbase, BINS2)], h_rd)
        d_ai.wait()

        @pl.loop(0, BINS2 // LANES, unroll=8)
        def _(i):
            sl = pl.ds(i * LANES, LANES)
            nd16 = h_rd[sl]
            ai16 = ai_in_v[sl]
            aio_v[sl] = jnp.where(nd16 > z16, zu16, ai16 + ou16)

        pltpu.sync_copy(aio_v, ai_out.at[pl.ds(gbase, BINS2)])


_sc_hist = pl.kernel(
    _sc_body,
    out_type=[
        jax.ShapeDtypeStruct((N_FEATURES,), jnp.float32),
        jax.ShapeDtypeStruct((N_FEATURES,), jnp.uint32),
    ],
    mesh=plsc.VectorSubcoreMesh(core_axis_name="c", subcore_axis_name="s"),
    scratch_types=[
        pltpu.VMEM_SHARED((N_FEATURES,), jnp.int32),
        pltpu.VMEM((PER_TILE_IDX,), jnp.int32),
        pltpu.VMEM((PER_TILE_IDX,), jnp.float32),
        pltpu.VMEM((PER_TILE_IDX,), jnp.int32),
        pltpu.VMEM((PER_TILE_IDX,), jnp.int32),
        pltpu.VMEM((BINS2,), jnp.float32),
        pltpu.VMEM((BINS2,), jnp.uint32),
        pltpu.VMEM((BINS2,), jnp.int32),
        pltpu.VMEM((BINS2,), jnp.float32),
        pltpu.VMEM((BINS2,), jnp.uint32),
        pltpu.VMEM((2, LANES), jnp.float32),
        pltpu.SemaphoreType.DMA,
        pltpu.SemaphoreType.DMA,
        pltpu.SemaphoreType.DMA,
        pltpu.SemaphoreType.DMA,
        pltpu.SemaphoreType.DMA,
        pltpu.SemaphoreType.DMA,
    ],
)


def kernel(x, k_weights, k_indices, feature_density, activated_in,
           feature_means, feature_square_means, avg_norm, n_steps):
    wf = (n_steps / (n_steps + 1)).astype(jnp.float32)
    nwf = (1 / (n_steps + 1)).astype(jnp.float32)

    params_tc = jnp.stack([wf, nwf / BATCH, avg_norm])
    means2, sq2, norm11 = _tc_stats(params_tc, x, feature_means,
                                    feature_square_means)

    idx3 = k_indices.reshape(TOTAL_IDX)
    w3 = k_weights.reshape(TOTAL_IDX)

    prm2 = jnp.stack([jnp.full((LANES,), wf, dtype=jnp.float32),
                      jnp.full((LANES,), nwf / BATCH, dtype=jnp.float32)])
    ones_host = jnp.ones((PER_TILE_IDX,), dtype=jnp.int32)
    zeros_host = jnp.zeros((ZEROS_N,), dtype=jnp.int32)
    fd_new, ai_new = _sc_hist(prm2, idx3, w3, feature_density, activated_in,
                              ones_host, zeros_host)

    return (norm11[0, 0], means2.reshape(D_MODEL), sq2.reshape(D_MODEL),
            fd_new, ai_new)


# speedup vs baseline: 64.4154x; 1.0286x over previous
import functools

import jax
import jax.numpy as jnp
from jax import lax
from jax.experimental import pallas as pl
from jax.experimental.pallas import tpu as pltpu
from jax.experimental.pallas import tpu_sc as plsc

D_MODEL = 1024
N_FEATURES = 131072
BATCH = 4096
K = 64
TOTAL_IDX = BATCH * K
DEATH_THRESHOLD = 0.01

NC = 2
NS = 16
LANES = 16
PER_TILE_IDX = TOTAL_IDX // NS
BINS2 = N_FEATURES // NS
ZEROS_N = N_FEATURES // NS

_NBLK = 8
_BLK = BATCH // _NBLK


def _tc_body(params, x_ref, fm_ref, fsm_ref, means_out, sq_out, norm_out):
    i = pl.program_id(0)
    blk = x_ref[...]
    sq = blk * blk
    csum = jnp.sum(blk, axis=0, keepdims=True)
    cssum = jnp.sum(sq, axis=0, keepdims=True)
    nsum = jnp.sum(jnp.sqrt(jnp.sum(sq, axis=1, keepdims=True)))

    @pl.when(i == 0)
    def _():
        means_out[...] = csum
        sq_out[...] = cssum
        norm_out[0, 0] = nsum

    @pl.when(i > 0)
    def _():
        means_out[...] += csum
        sq_out[...] += cssum
        norm_out[0, 0] += nsum

    @pl.when(i == _NBLK - 1)
    def _():
        wf = params[0]
        scale = params[1]
        means_out[...] = fm_ref[...] * wf + means_out[...] * scale
        sq_out[...] = fsm_ref[...] * wf + sq_out[...] * scale
        norm_out[0, 0] = params[2] * wf + norm_out[0, 0] * scale


def _tc_stats(params_tc, x, feature_means, feature_square_means):
    return pl.pallas_call(
        _tc_body,
        grid=(_NBLK,),
        in_specs=[
            pl.BlockSpec(memory_space=pltpu.SMEM),
            pl.BlockSpec((_BLK, D_MODEL), lambda i: (i, 0)),
            pl.BlockSpec((1, D_MODEL), lambda i: (0, 0)),
            pl.BlockSpec((1, D_MODEL), lambda i: (0, 0)),
        ],
        out_specs=[
            pl.BlockSpec((1, D_MODEL), lambda i: (0, 0)),
            pl.BlockSpec((1, D_MODEL), lambda i: (0, 0)),
            pl.BlockSpec(memory_space=pltpu.SMEM),
        ],
        out_shape=[
            jax.ShapeDtypeStruct((1, D_MODEL), jnp.float32),
            jax.ShapeDtypeStruct((1, D_MODEL), jnp.float32),
            jax.ShapeDtypeStruct((1, 1), jnp.float32),
        ],
        compiler_params=pltpu.CompilerParams(
            dimension_semantics=("arbitrary",)),
    )(params_tc, x, feature_means.reshape(1, D_MODEL),
      feature_square_means.reshape(1, D_MODEL))


def _sc_body(prm_hbm, idx_hbm, w_hbm, fd_hbm, ai_hbm, zeros_hbm,
             fd_out, ai_out,
             hist,
             idx_v, w_v, nd_v, ones_v,
             fd_in_v, ai_in_v, h_rd, fdo_v, aio_v, prm_v,
             semi, semw, semz, semf, sem0):
    c = lax.axis_index("c")
    s = lax.axis_index("s")

    ibase = s * PER_TILE_IDX
    d_idx = pltpu.async_copy(idx_hbm.at[pl.ds(ibase, PER_TILE_IDX)], idx_v,
                             semi)
    zbase = s * ZEROS_N
    d_z = pltpu.async_copy(zeros_hbm, hist.at[pl.ds(zbase, ZEROS_N)], semz)
    gbase = s * BINS2

    one16 = jnp.full((LANES,), 1, dtype=jnp.int32)
    z16 = jnp.zeros((LANES,), dtype=jnp.int32)
    thr16 = jnp.full((LANES,), DEATH_THRESHOLD, dtype=jnp.float32)
    zu16 = jnp.zeros((LANES,), dtype=jnp.uint32)
    ou16 = jnp.full((LANES,), 1, dtype=jnp.uint32)

    @pl.when(c == 0)
    def _():
        d_fd = pltpu.async_copy(fd_hbm.at[pl.ds(gbase, BINS2)], fd_in_v,
                                semf)
        pltpu.sync_copy(prm_hbm, prm_v)

        @pl.loop(0, PER_TILE_IDX // LANES, unroll=8)
        def _(j):
            ones_v[pl.ds(j * LANES, LANES)] = one16

        d_z.wait()
        plsc.subcore_barrier()
        d_idx.wait()
        pltpu.async_copy(ones_v, hist.at[idx_v], sem0, add=True).wait()
        plsc.subcore_barrier()
        pltpu.sync_copy(hist.at[pl.ds(gbase, BINS2)], h_rd)
        d_fd.wait()
        wf16 = prm_v[0, :]
        scale16 = prm_v[1, :]

        @pl.loop(0, BINS2 // LANES, unroll=8)
        def _(i):
            sl = pl.ds(i * LANES, LANES)
            cnt16 = h_rd[sl].astype(jnp.float32)
            fdo_v[sl] = fd_in_v[sl] * wf16 + cnt16 * scale16

        pltpu.sync_copy(fdo_v, fd_out.at[pl.ds(gbase, BINS2)])

    @pl.when(c == 1)
    def _():
        d_w = pltpu.async_copy(w_hbm.at[pl.ds(ibase, PER_TILE_IDX)], w_v,
                               semw)
        d_ai = pltpu.async_copy(ai_hbm.at[pl.ds(gbase, BINS2)], ai_in_v,
                                semf)
        d_w.wait()

        @pl.loop(0, PER_TILE_IDX // LANES, unroll=8)
        def _(j):
            sl = pl.ds(j * LANES, LANES)
            w16 = w_v[sl]
            nd_v[sl] = jnp.where(w16 > thr16, one16, z16)

        d_z.wait()
        plsc.subcore_barrier()
        d_idx.wait()
        pltpu.async_copy(nd_v, hist.at[idx_v], sem0, add=True).wait()
        plsc.subcore_barrier()
        pltpu.sync_copy(hist.at[pl.ds(gbase, BINS2)], h_rd)
        d_ai.wait()

        @pl.loop(0, BINS2 // LANES, unroll=8)
        def _(i):
            sl = pl.ds(i * LANES, LANES)
            nd16 = h_rd[sl]
            ai16 = ai_in_v[sl]
            aio_v[sl] = jnp.where(nd16 > z16, zu16, ai16 + ou16)

        pltpu.sync_copy(aio_v, ai_out.at[pl.ds(gbase, BINS2)])


_sc_hist = pl.kernel(
    _sc_body,
    out_type=[
        jax.ShapeDtypeStruct((N_FEATURES,), jnp.float32),
        jax.ShapeDtypeStruct((N_FEATURES,), jnp.uint32),
    ],
    mesh=plsc.VectorSubcoreMesh(core_axis_name="c", subcore_axis_name="s"),
    scratch_types=[
        pltpu.VMEM_SHARED((N_FEATURES,), jnp.int32),
        pltpu.VMEM((PER_TILE_IDX,), jnp.int32),
        pltpu.VMEM((PER_TILE_IDX,), jnp.float32),
        pltpu.VMEM((PER_TILE_IDX,), jnp.int32),
        pltpu.VMEM((PER_TILE_IDX,), jnp.int32),
        pltpu.VMEM((BINS2,), jnp.float32),
        pltpu.VMEM((BINS2,), jnp.uint32),
        pltpu.VMEM((BINS2,), jnp.int32),
        pltpu.VMEM((BINS2,), jnp.float32),
        pltpu.VMEM((BINS2,), jnp.uint32),
        pltpu.VMEM((2, LANES), jnp.float32),
        pltpu.SemaphoreType.DMA,
        pltpu.SemaphoreType.DMA,
        pltpu.SemaphoreType.DMA,
        pltpu.SemaphoreType.DMA,
        pltpu.SemaphoreType.DMA,
    ],
)


def kernel(x, k_weights, k_indices, feature_density, activated_in,
           feature_means, feature_square_means, avg_norm, n_steps):
    wf = (n_steps / (n_steps + 1)).astype(jnp.float32)
    nwf = (1 / (n_steps + 1)).astype(jnp.float32)

    params_tc = jnp.stack([wf, nwf / BATCH, avg_norm])
    means2, sq2, norm11 = _tc_stats(params_tc, x, feature_means,
                                    feature_square_means)

    prm2 = jnp.stack([jnp.full((LANES,), wf, dtype=jnp.float32),
                      jnp.full((LANES,), nwf / BATCH, dtype=jnp.float32)])
    zeros_host = jnp.zeros((ZEROS_N,), dtype=jnp.int32)
    fd_new, ai_new = _sc_hist(prm2, k_indices.reshape(TOTAL_IDX),
                              k_weights.reshape(TOTAL_IDX), feature_density,
                              activated_in, zeros_host)

    return (norm11[0, 0], means2.reshape(D_MODEL), sq2.reshape(D_MODEL),
            fd_new, ai_new)


# speedup vs baseline: 70.0125x; 1.0869x over previous
import functools

import jax
import jax.numpy as jnp
from jax import lax
from jax.experimental import pallas as pl
from jax.experimental.pallas import tpu as pltpu
from jax.experimental.pallas import tpu_sc as plsc

D_MODEL = 1024
N_FEATURES = 131072
BATCH = 4096
K = 64
TOTAL_IDX = BATCH * K
DEATH_THRESHOLD = 0.01

NC = 2
NS = 16
LANES = 16
PER_TILE_IDX = TOTAL_IDX // NS
BINS2 = N_FEATURES // NS
ZEROS_N = N_FEATURES // NS

CH = 4
CHN = PER_TILE_IDX // CH

_NBLK = 8
_BLK = BATCH // _NBLK


def _tc_body(params, x_ref, fm_ref, fsm_ref, means_out, sq_out, norm_out):
    i = pl.program_id(0)
    blk = x_ref[...]
    sq = blk * blk
    csum = jnp.sum(blk, axis=0, keepdims=True)
    cssum = jnp.sum(sq, axis=0, keepdims=True)
    nsum = jnp.sum(jnp.sqrt(jnp.sum(sq, axis=1, keepdims=True)))

    @pl.when(i == 0)
    def _():
        means_out[...] = csum
        sq_out[...] = cssum
        norm_out[0, 0] = nsum

    @pl.when(i > 0)
    def _():
        means_out[...] += csum
        sq_out[...] += cssum
        norm_out[0, 0] += nsum

    @pl.when(i == _NBLK - 1)
    def _():
        wf = params[0]
        scale = params[1]
        means_out[...] = fm_ref[...] * wf + means_out[...] * scale
        sq_out[...] = fsm_ref[...] * wf + sq_out[...] * scale
        norm_out[0, 0] = params[2] * wf + norm_out[0, 0] * scale


def _tc_stats(params_tc, x, feature_means, feature_square_means):
    return pl.pallas_call(
        _tc_body,
        grid=(_NBLK,),
        in_specs=[
            pl.BlockSpec(memory_space=pltpu.SMEM),
            pl.BlockSpec((_BLK, D_MODEL), lambda i: (i, 0)),
            pl.BlockSpec((1, D_MODEL), lambda i: (0, 0)),
            pl.BlockSpec((1, D_MODEL), lambda i: (0, 0)),
        ],
        out_specs=[
            pl.BlockSpec((1, D_MODEL), lambda i: (0, 0)),
            pl.BlockSpec((1, D_MODEL), lambda i: (0, 0)),
            pl.BlockSpec(memory_space=pltpu.SMEM),
        ],
        out_shape=[
            jax.ShapeDtypeStruct((1, D_MODEL), jnp.float32),
            jax.ShapeDtypeStruct((1, D_MODEL), jnp.float32),
            jax.ShapeDtypeStruct((1, 1), jnp.float32),
        ],
        compiler_params=pltpu.CompilerParams(
            dimension_semantics=("arbitrary",)),
    )(params_tc, x, feature_means.reshape(1, D_MODEL),
      feature_square_means.reshape(1, D_MODEL))


def _sc_body(ns_hbm, idx_hbm, w_hbm, fd_hbm, ai_hbm, zeros_hbm,
             fd_out, ai_out,
             hist,
             idx_v0, idx_v1, idx_v2, idx_v3,
             w_v0, w_v1, w_v2, w_v3,
             nd_v0, nd_v1, nd_v2, nd_v3,
             ones_v,
             fd_in_v, ai_in_v, h_rd, fdo_v, aio_v, prm_v,
             semi, semw, semz, semf, sem0):
    idx_vs = [idx_v0, idx_v1, idx_v2, idx_v3]
    w_vs = [w_v0, w_v1, w_v2, w_v3]
    nd_vs = [nd_v0, nd_v1, nd_v2, nd_v3]
    c = lax.axis_index("c")
    s = lax.axis_index("s")

    ibase = s * PER_TILE_IDX
    d_idx = [pltpu.async_copy(idx_hbm.at[pl.ds(ibase + k * CHN, CHN)],
                              idx_vs[k], semi) for k in range(CH)]
    zbase = s * ZEROS_N
    d_z = pltpu.async_copy(zeros_hbm, hist.at[pl.ds(zbase, ZEROS_N)], semz)
    gbase = s * BINS2

    one16 = jnp.full((LANES,), 1, dtype=jnp.int32)
    z16 = jnp.zeros((LANES,), dtype=jnp.int32)
    thr16 = jnp.full((LANES,), DEATH_THRESHOLD, dtype=jnp.float32)
    zu16 = jnp.zeros((LANES,), dtype=jnp.uint32)
    ou16 = jnp.full((LANES,), 1, dtype=jnp.uint32)
    onef16 = jnp.full((LANES,), 1.0, dtype=jnp.float32)
    inv4k16 = jnp.full((LANES,), 1.0 / BATCH, dtype=jnp.float32)

    @pl.when(c == 0)
    def _():
        d_fd = pltpu.async_copy(fd_hbm.at[pl.ds(gbase, BINS2)], fd_in_v,
                                semf)
        pltpu.sync_copy(ns_hbm, prm_v)
        ns16 = prm_v[...]
        nsf = ns16.astype(jnp.float32)
        np1 = (ns16 + ou16).astype(jnp.float32)
        wf16 = nsf / np1
        scale16 = (onef16 / np1) * inv4k16

        @pl.loop(0, CHN // LANES, unroll=8)
        def _(j):
            ones_v[pl.ds(j * LANES, LANES)] = one16

        d_z.wait()
        plsc.subcore_barrier()
        for d in d_idx:
            d.wait()
        descs = [pltpu.async_copy(ones_v, hist.at[idx_vs[k]], sem0,
                                  add=True) for k in range(CH)]
        for d in descs:
            d.wait()
        plsc.subcore_barrier()
        pltpu.sync_copy(hist.at[pl.ds(gbase, BINS2)], h_rd)
        d_fd.wait()

        @pl.loop(0, BINS2 // LANES, unroll=8)
        def _(i):
            sl = pl.ds(i * LANES, LANES)
            cnt16 = h_rd[sl].astype(jnp.float32)
            fdo_v[sl] = fd_in_v[sl] * wf16 + cnt16 * scale16

        pltpu.sync_copy(fdo_v, fd_out.at[pl.ds(gbase, BINS2)])

    @pl.when(c == 1)
    def _():
        d_w = [pltpu.async_copy(w_hbm.at[pl.ds(ibase + k * CHN, CHN)],
                                w_vs[k], semw) for k in range(CH)]
        d_ai = pltpu.async_copy(ai_hbm.at[pl.ds(gbase, BINS2)], ai_in_v,
                                semf)
        descs = []
        for k in range(CH):
            d_w[k].wait()

            wk, ndk = w_vs[k], nd_vs[k]

            @pl.loop(0, CHN // LANES, unroll=8)
            def _(j, wk=wk, ndk=ndk):
                sl = pl.ds(j * LANES, LANES)
                w16 = wk[sl]
                ndk[sl] = jnp.where(w16 > thr16, one16, z16)

            if k == 0:
                d_z.wait()
                plsc.subcore_barrier()
                for d in d_idx:
                    d.wait()
            descs.append(pltpu.async_copy(nd_vs[k], hist.at[idx_vs[k]],
                                          sem0, add=True))
        for d in descs:
            d.wait()
        plsc.subcore_barrier()
        pltpu.sync_copy(hist.at[pl.ds(gbase, BINS2)], h_rd)
        d_ai.wait()

        @pl.loop(0, BINS2 // LANES, unroll=8)
        def _(i):
            sl = pl.ds(i * LANES, LANES)
            nd16 = h_rd[sl]
            ai16 = ai_in_v[sl]
            aio_v[sl] = jnp.where(nd16 > z16, zu16, ai16 + ou16)

        pltpu.sync_copy(aio_v, ai_out.at[pl.ds(gbase, BINS2)])


_sc_hist = pl.kernel(
    _sc_body,
    out_type=[
        jax.ShapeDtypeStruct((N_FEATURES,), jnp.float32),
        jax.ShapeDtypeStruct((N_FEATURES,), jnp.uint32),
    ],
    mesh=plsc.VectorSubcoreMesh(core_axis_name="c", subcore_axis_name="s"),
    scratch_types=[
        pltpu.VMEM_SHARED((N_FEATURES,), jnp.int32),
        pltpu.VMEM((CHN,), jnp.int32),
        pltpu.VMEM((CHN,), jnp.int32),
        pltpu.VMEM((CHN,), jnp.int32),
        pltpu.VMEM((CHN,), jnp.int32),
        pltpu.VMEM((CHN,), jnp.float32),
        pltpu.VMEM((CHN,), jnp.float32),
        pltpu.VMEM((CHN,), jnp.float32),
        pltpu.VMEM((CHN,), jnp.float32),
        pltpu.VMEM((CHN,), jnp.int32),
        pltpu.VMEM((CHN,), jnp.int32),
        pltpu.VMEM((CHN,), jnp.int32),
        pltpu.VMEM((CHN,), jnp.int32),
        pltpu.VMEM((CHN,), jnp.int32),
        pltpu.VMEM((BINS2,), jnp.float32),
        pltpu.VMEM((BINS2,), jnp.uint32),
        pltpu.VMEM((BINS2,), jnp.int32),
        pltpu.VMEM((BINS2,), jnp.float32),
        pltpu.VMEM((BINS2,), jnp.uint32),
        pltpu.VMEM((LANES,), jnp.uint32),
        pltpu.SemaphoreType.DMA,
        pltpu.SemaphoreType.DMA,
        pltpu.SemaphoreType.DMA,
        pltpu.SemaphoreType.DMA,
        pltpu.SemaphoreType.DMA,
    ],
)


def kernel(x, k_weights, k_indices, feature_density, activated_in,
           feature_means, feature_square_means, avg_norm, n_steps):
    wf = (n_steps / (n_steps + 1)).astype(jnp.float32)
    nwf = (1 / (n_steps + 1)).astype(jnp.float32)

    params_tc = jnp.stack([wf, nwf / BATCH, avg_norm])
    means2, sq2, norm11 = _tc_stats(params_tc, x, feature_means,
                                    feature_square_means)

    ns16_host = jnp.full((LANES,), n_steps, dtype=jnp.uint32)
    zeros_host = jnp.zeros((ZEROS_N,), dtype=jnp.int32)
    fd_new, ai_new = _sc_hist(ns16_host, k_indices.reshape(TOTAL_IDX),
                              k_weights.reshape(TOTAL_IDX), feature_density,
                              activated_in, zeros_host)

    return (norm11[0, 0], means2.reshape(D_MODEL), sq2.reshape(D_MODEL),
            fd_new, ai_new)


# speedup vs baseline: 70.3001x; 1.0041x over previous
import functools

import jax
import jax.numpy as jnp
from jax import lax
from jax.experimental import pallas as pl
from jax.experimental.pallas import tpu as pltpu
from jax.experimental.pallas import tpu_sc as plsc

D_MODEL = 1024
N_FEATURES = 131072
BATCH = 4096
K = 64
TOTAL_IDX = BATCH * K
DEATH_THRESHOLD = 0.01

NC = 2
NS = 16
LANES = 16
PER_TILE_IDX = TOTAL_IDX // NS
BINS2 = N_FEATURES // NS
ZEROS_N = N_FEATURES // NS

CH = 4
CHN = PER_TILE_IDX // CH

_NBLK = 8
_BLK = BATCH // _NBLK


def _tc_body(params, x_ref, fm_ref, fsm_ref, means_out, sq_out, norm_out):
    i = pl.program_id(0)
    blk = x_ref[...]
    sq = blk * blk
    csum = jnp.sum(blk, axis=0, keepdims=True)
    cssum = jnp.sum(sq, axis=0, keepdims=True)
    nsum = jnp.sum(jnp.sqrt(jnp.sum(sq, axis=1, keepdims=True)))

    @pl.when(i == 0)
    def _():
        means_out[...] = csum
        sq_out[...] = cssum
        norm_out[0, 0] = nsum

    @pl.when(i > 0)
    def _():
        means_out[...] += csum
        sq_out[...] += cssum
        norm_out[0, 0] += nsum

    @pl.when(i == _NBLK - 1)
    def _():
        wf = params[0]
        scale = params[1]
        means_out[...] = fm_ref[...] * wf + means_out[...] * scale
        sq_out[...] = fsm_ref[...] * wf + sq_out[...] * scale
        norm_out[0, 0] = params[2] * wf + norm_out[0, 0] * scale


def _tc_stats(params_tc, x, feature_means, feature_square_means):
    return pl.pallas_call(
        _tc_body,
        grid=(_NBLK,),
        in_specs=[
            pl.BlockSpec(memory_space=pltpu.SMEM),
            pl.BlockSpec((_BLK, D_MODEL), lambda i: (i, 0)),
            pl.BlockSpec((1, D_MODEL), lambda i: (0, 0)),
            pl.BlockSpec((1, D_MODEL), lambda i: (0, 0)),
        ],
        out_specs=[
            pl.BlockSpec((1, D_MODEL), lambda i: (0, 0)),
            pl.BlockSpec((1, D_MODEL), lambda i: (0, 0)),
            pl.BlockSpec(memory_space=pltpu.SMEM),
        ],
        out_shape=[
            jax.ShapeDtypeStruct((1, D_MODEL), jnp.float32),
            jax.ShapeDtypeStruct((1, D_MODEL), jnp.float32),
            jax.ShapeDtypeStruct((1, 1), jnp.float32),
        ],
        compiler_params=pltpu.CompilerParams(
            dimension_semantics=("arbitrary",)),
    )(params_tc, x, feature_means.reshape(1, D_MODEL),
      feature_square_means.reshape(1, D_MODEL))


def _sc_body(idx_hbm, w_hbm, zeros_hbm,
             cnt_out, nd_out,
             hist,
             idx_v0, idx_v1, idx_v2, idx_v3,
             w_v0, w_v1, w_v2, w_v3,
             nd_v0, nd_v1, nd_v2, nd_v3,
             ones_v,
             semi, semw, semz, sem0):
    idx_vs = [idx_v0, idx_v1, idx_v2, idx_v3]
    w_vs = [w_v0, w_v1, w_v2, w_v3]
    nd_vs = [nd_v0, nd_v1, nd_v2, nd_v3]
    c = lax.axis_index("c")
    s = lax.axis_index("s")

    ibase = s * PER_TILE_IDX
    d_idx = [pltpu.async_copy(idx_hbm.at[pl.ds(ibase + k * CHN, CHN)],
                              idx_vs[k], semi) for k in range(CH)]
    zbase = s * ZEROS_N
    d_z = pltpu.async_copy(zeros_hbm, hist.at[pl.ds(zbase, ZEROS_N)], semz)
    gbase = s * BINS2

    one16 = jnp.full((LANES,), 1, dtype=jnp.int32)
    z16 = jnp.zeros((LANES,), dtype=jnp.int32)
    thr16 = jnp.full((LANES,), DEATH_THRESHOLD, dtype=jnp.float32)

    @pl.when(c == 0)
    def _():
        @pl.loop(0, CHN // LANES, unroll=8)
        def _(j):
            ones_v[pl.ds(j * LANES, LANES)] = one16

        d_z.wait()
        plsc.subcore_barrier()
        for d in d_idx:
            d.wait()
        descs = [pltpu.async_copy(ones_v, hist.at[idx_vs[k]], sem0,
                                  add=True) for k in range(CH)]
        for d in descs:
            d.wait()
        plsc.subcore_barrier()
        pltpu.sync_copy(hist.at[pl.ds(gbase, BINS2)],
                        cnt_out.at[pl.ds(gbase, BINS2)])

    @pl.when(c == 1)
    def _():
        d_w = [pltpu.async_copy(w_hbm.at[pl.ds(ibase + k * CHN, CHN)],
                                w_vs[k], semw) for k in range(CH)]
        descs = []
        for k in range(CH):
            d_w[k].wait()
            wk, ndk = w_vs[k], nd_vs[k]

            @pl.loop(0, CHN // LANES, unroll=8)
            def _(j, wk=wk, ndk=ndk):
                sl = pl.ds(j * LANES, LANES)
                w16 = wk[sl]
                ndk[sl] = jnp.where(w16 > thr16, one16, z16)

            if k == 0:
                d_z.wait()
                plsc.subcore_barrier()
                for d in d_idx:
                    d.wait()
            descs.append(pltpu.async_copy(nd_vs[k], hist.at[idx_vs[k]],
                                          sem0, add=True))
        for d in descs:
            d.wait()
        plsc.subcore_barrier()
        pltpu.sync_copy(hist.at[pl.ds(gbase, BINS2)],
                        nd_out.at[pl.ds(gbase, BINS2)])


_sc_hist = pl.kernel(
    _sc_body,
    out_type=[
        jax.ShapeDtypeStruct((N_FEATURES,), jnp.int32),
        jax.ShapeDtypeStruct((N_FEATURES,), jnp.int32),
    ],
    mesh=plsc.VectorSubcoreMesh(core_axis_name="c", subcore_axis_name="s"),
    scratch_types=[
        pltpu.VMEM_SHARED((N_FEATURES,), jnp.int32),
        pltpu.VMEM((CHN,), jnp.int32),
        pltpu.VMEM((CHN,), jnp.int32),
        pltpu.VMEM((CHN,), jnp.int32),
        pltpu.VMEM((CHN,), jnp.int32),
        pltpu.VMEM((CHN,), jnp.float32),
        pltpu.VMEM((CHN,), jnp.float32),
        pltpu.VMEM((CHN,), jnp.float32),
        pltpu.VMEM((CHN,), jnp.float32),
        pltpu.VMEM((CHN,), jnp.int32),
        pltpu.VMEM((CHN,), jnp.int32),
        pltpu.VMEM((CHN,), jnp.int32),
        pltpu.VMEM((CHN,), jnp.int32),
        pltpu.VMEM((CHN,), jnp.int32),
        pltpu.SemaphoreType.DMA,
        pltpu.SemaphoreType.DMA,
        pltpu.SemaphoreType.DMA,
        pltpu.SemaphoreType.DMA,
    ],
)


def _upd_body(params, fd_ref, ai_ref, cnt_ref, nd_ref, fd_out, ai_out):
    wf = params[0]
    scale = params[1]
    cnt = cnt_ref[...].astype(jnp.float32)
    fd_out[...] = fd_ref[...] * wf + cnt * scale
    nd = nd_ref[...]
    ai_out[...] = jnp.where(nd > 0, jnp.uint32(0), ai_ref[...] + jnp.uint32(1))


def _tc_update(params_tc, feature_density, activated_in, cnt_h, nd_h):
    return pl.pallas_call(
        _upd_body,
        in_specs=[
            pl.BlockSpec(memory_space=pltpu.SMEM),
            pl.BlockSpec((N_FEATURES,), lambda: (0,)),
            pl.BlockSpec((N_FEATURES,), lambda: (0,)),
            pl.BlockSpec((N_FEATURES,), lambda: (0,)),
            pl.BlockSpec((N_FEATURES,), lambda: (0,)),
        ],
        out_specs=[
            pl.BlockSpec((N_FEATURES,), lambda: (0,)),
            pl.BlockSpec((N_FEATURES,), lambda: (0,)),
        ],
        out_shape=[
            jax.ShapeDtypeStruct((N_FEATURES,), jnp.float32),
            jax.ShapeDtypeStruct((N_FEATURES,), jnp.uint32),
        ],
    )(params_tc, feature_density, activated_in, cnt_h, nd_h)


def kernel(x, k_weights, k_indices, feature_density, activated_in,
           feature_means, feature_square_means, avg_norm, n_steps):
    wf = (n_steps / (n_steps + 1)).astype(jnp.float32)
    nwf = (1 / (n_steps + 1)).astype(jnp.float32)

    params_tc = jnp.stack([wf, nwf / BATCH, avg_norm])
    means2, sq2, norm11 = _tc_stats(params_tc, x, feature_means,
                                    feature_square_means)

    zeros_host = jnp.zeros((ZEROS_N,), dtype=jnp.int32)
    cnt_h, nd_h = _sc_hist(k_indices.reshape(TOTAL_IDX),
                           k_weights.reshape(TOTAL_IDX), zeros_host)
    fd_new, ai_new = _tc_update(params_tc, feature_density, activated_in,
                                cnt_h, nd_h)

    return (norm11[0, 0], means2.reshape(D_MODEL), sq2.reshape(D_MODEL),
            fd_new, ai_new)
